# baseline (speedup 1.0000x reference)
import functools

import jax
import jax.numpy as jnp
from jax import lax
from jax.experimental import pallas as pl
from jax.experimental.pallas import tpu as pltpu
from jax.experimental.pallas import tpu_sc as plsc

_N = 8192
_A = 1.1201
_B = 0.7990

_NC = 2
_NS = 16
_NW = _NC * _NS
_TOTAL = 16384
_BPW = _TOTAL // _NW
_CHUNK = 128
_NCHUNK = _BPW // _CHUNK


@functools.lru_cache(maxsize=None)
def _make_sc_gather():
    mesh = plsc.VectorSubcoreMesh(core_axis_name="c", subcore_axis_name="s")

    @functools.partial(
        pl.kernel,
        mesh=mesh,
        out_type=jax.ShapeDtypeStruct((_TOTAL,), jnp.float32),
        scratch_types=[
            pltpu.VMEM((_BPW,), jnp.int32),
            pltpu.VMEM((_BPW,), jnp.int32),
            pltpu.VMEM((_NCHUNK, _CHUNK), jnp.int32),
            pltpu.VMEM((_NCHUNK, _CHUNK), jnp.float32),
            pltpu.SemaphoreType.DMA,
        ],
    )
    def _sc_gather(flat_hbm, ind1_hbm, ind2_hbm, out_hbm, i1_v, i2_v, flat_v,
                   vals_v, sem):
        wid = lax.axis_index("s") * _NC + lax.axis_index("c")
        base = wid * _BPW
        pltpu.sync_copy(ind1_hbm.at[pl.ds(base, _BPW)], i1_v)
        pltpu.sync_copy(ind2_hbm.at[pl.ds(base, _BPW)], i2_v)
        for j in range(_NCHUNK):
            for k in range(_CHUNK // 16):
                sl = pl.ds(j * _CHUNK + k * 16, 16)
                flat_v[j, pl.ds(k * 16, 16)] = i1_v[sl] * _N + i2_v[sl]
        copies = []
        for j in range(_NCHUNK):
            copies.append(
                pltpu.async_copy(flat_hbm.at[flat_v.at[j]], vals_v.at[j], sem))
        for j in range(_NCHUNK):
            copies[j].wait()
            pltpu.sync_copy(vals_v.at[j], out_hbm.at[pl.ds(base + j * _CHUNK,
                                                           _CHUNK)])

    return _sc_gather


def _lowdim_body(x1_ref, y1_ref, x2_ref, y2_ref, out_ref):
    dx = x1_ref[...] - x2_ref[...]
    dy = y1_ref[...] - y2_ref[...]
    s = dx * dx + dy * dy
    powed = jnp.exp(_B * jnp.log(s))
    out_ref[...] = 1.0 / (1.0 + _A * powed)


def kernel(p1, p2, ind1, ind2, sym_prob):
    flat = sym_prob.reshape(-1)
    high = _make_sc_gather()(flat, ind1.astype(jnp.int32),
                             ind2.astype(jnp.int32))

    x1 = p1[:, 0].reshape(128, 128)
    y1 = p1[:, 1].reshape(128, 128)
    x2 = p2[:, 0].reshape(128, 128)
    y2 = p2[:, 1].reshape(128, 128)
    low = pl.pallas_call(
        _lowdim_body,
        out_shape=jax.ShapeDtypeStruct((128, 128), jnp.float32),
    )(x1, y1, x2, y2).reshape(_TOTAL)
    return (low, high)


# speedup vs baseline: 8.1284x; 8.1284x over previous
import functools

import jax
import jax.numpy as jnp
from jax import lax
from jax.experimental import pallas as pl
from jax.experimental.pallas import tpu as pltpu
from jax.experimental.pallas import tpu_sc as plsc

_N = 8192
_A = 1.1201
_B = 0.7990

_NC = 2
_NS = 16
_NW = _NC * _NS
_TOTAL = 16384
_BPW = _TOTAL // _NW
_CHUNK = 128
_NCHUNK = _BPW // _CHUNK


@functools.lru_cache(maxsize=None)
def _make_sc_gather():
    mesh = plsc.VectorSubcoreMesh(core_axis_name="c", subcore_axis_name="s")

    @functools.partial(
        pl.kernel,
        mesh=mesh,
        out_type=jax.ShapeDtypeStruct((_TOTAL,), jnp.float32),
        compiler_params=pltpu.CompilerParams(needs_layout_passes=False),
        scratch_types=[
            pltpu.VMEM((_BPW,), jnp.int32),
            pltpu.VMEM((_BPW,), jnp.int32),
            pltpu.VMEM((_NCHUNK, _CHUNK), jnp.int32),
            pltpu.VMEM((_BPW,), jnp.int32),
            pltpu.VMEM((_BPW, 128), jnp.float32),
            pltpu.VMEM((_BPW,), jnp.float32),
            pltpu.SemaphoreType.DMA,
        ],
    )
    def _sc_gather(lines_hbm, ind1_hbm, ind2_hbm, out_hbm, i1_v, i2_v, q_v,
                   lane_v, vals_v, out_v, sem):
        wid = lax.axis_index("s") * _NC + lax.axis_index("c")
        base = wid * _BPW
        pltpu.sync_copy(ind1_hbm.at[pl.ds(base, _BPW)], i1_v)
        pltpu.sync_copy(ind2_hbm.at[pl.ds(base, _BPW)], i2_v)
        for j in range(_NCHUNK):
            for k in range(_CHUNK // 16):
                sl = pl.ds(j * _CHUNK + k * 16, 16)
                row = i1_v[sl]
                col = i2_v[sl]
                q_v[j, pl.ds(k * 16, 16)] = \
                    ((row >> 3) * (_N // 128) + (col >> 7)) * 8 + (row & 7)
                lane_v[sl] = col & 127
        copies = []
        for j in range(_NCHUNK):
            copies.append(
                pltpu.async_copy(lines_hbm.at[q_v.at[j]],
                                 vals_v.at[pl.ds(j * _CHUNK, _CHUNK)], sem))
        lane16 = lax.iota(jnp.int32, 16)
        for j in range(_NCHUNK):
            copies[j].wait()
            for k in range(_CHUNK // 16):
                sl = pl.ds(j * _CHUNK + k * 16, 16)
                rows16 = lane16 + (j * _CHUNK + k * 16)
                out_v[sl] = plsc.load_gather(vals_v, [rows16, lane_v[sl]])
        pltpu.sync_copy(out_v, out_hbm.at[pl.ds(base, _BPW)])

    return _sc_gather


def _lowdim_body(x1_ref, y1_ref, x2_ref, y2_ref, out_ref):
    dx = x1_ref[...] - x2_ref[...]
    dy = y1_ref[...] - y2_ref[...]
    s = dx * dx + dy * dy
    powed = jnp.exp(_B * jnp.log(s))
    out_ref[...] = 1.0 / (1.0 + _A * powed)


def kernel(p1, p2, ind1, ind2, sym_prob):
    lines = sym_prob.reshape(1024, 8, 64, 128).transpose(0, 2, 1, 3) \
        .reshape(_N * _N // 128, 128)
    high = _make_sc_gather()(lines, ind1.astype(jnp.int32),
                             ind2.astype(jnp.int32))

    x1 = p1[:, 0].reshape(128, 128)
    y1 = p1[:, 1].reshape(128, 128)
    x2 = p2[:, 0].reshape(128, 128)
    y2 = p2[:, 1].reshape(128, 128)
    low = pl.pallas_call(
        _lowdim_body,
        out_shape=jax.ShapeDtypeStruct((128, 128), jnp.float32),
    )(x1, y1, x2, y2).reshape(_TOTAL)
    return (low, high)
